# baseline (speedup 1.0000x reference)
import jax
import jax.numpy as jnp
from jax.experimental import pallas as pl

_NUM_CLUSTERS = 8
_EPS = 1e-05
_BLOCK_ROWS = 2048

_SQRT_2_OVER_PI = 0.7978845608028654
_LOG2E = 1.4426950408889634
_BIG = 3.0e38


def _body(x_ref, c_ref, c2_ref, o_ref):
    xb = x_ref[...]
    caug = c_ref[...]
    d = xb.shape[1]

    cdims = (((1,), (1,)), ((), ()))
    dots = jax.lax.dot_general(
        xb, caug, cdims, preferred_element_type=jnp.float32
    )
    s1 = jnp.sum(xb, axis=1, keepdims=True)
    s2 = jnp.sum(xb * xb, axis=1, keepdims=True)

    dist = s2 - 2.0 * dots + c2_ref[...]

    mind = jnp.min(dist, axis=1, keepdims=True)
    lane = jax.lax.broadcasted_iota(jnp.int32, dist.shape, 1)
    lab = jnp.min(
        jnp.where(dist == mind, lane, _NUM_CLUSTERS), axis=1, keepdims=True
    )

    mean = s1 * (1.0 / d)
    var = (s2 - s1 * mean) * (1.0 / (d - 1))
    rstd = jax.lax.rsqrt(var + _EPS)
    xn = (xb - mean) * rstd

    ha = jnp.where(
        lab == 1, _SQRT_2_OVER_PI,
        jnp.where(lab == 2, 1.0, jnp.where((lab == 3) | (lab == 4), 0.5, 0.0)),
    )
    hb = jnp.where(lab == 1, _SQRT_2_OVER_PI * 0.044715, 0.0)
    fp = jnp.where((lab == 1) | (lab == 3), 0.5, 0.0)
    fq = jnp.where(lab == 2, 1.0, jnp.where(lab == 4, 0.5, 0.0))
    ft = jnp.where(lab == 2, -1.0, 0.0)
    is_sp = lab == 7
    is_sig = (lab >= 1) & (lab <= 4)
    lo = jnp.where(is_sp, -_BIG, 0.0)
    up = jnp.where(lab == 5, 6.0, _BIG)
    ek = jnp.where(lab == 6, _LOG2E, jnp.where(is_sp, -_LOG2E, 0.0))

    xnsq = xn * xn
    th = jnp.tanh(xn * (ha + hb * xnsq))
    sig_out = (fp * xn + fq) * (th + 1.0) + ft

    base = jnp.minimum(jnp.maximum(xn, lo), up)
    em = jnp.exp2(ek * xn)
    extra = jnp.where(is_sp, jnp.log(1.0 + em), jnp.minimum(em - 1.0, 0.0))

    out = jnp.where(is_sig, sig_out, base + extra)
    o_ref[...] = out


@jax.jit
def kernel(x, centroids):
    n, d = x.shape
    c2 = jnp.sum(centroids * centroids, axis=-1)[None, :]
    grid = (n // _BLOCK_ROWS,)
    return pl.pallas_call(
        _body,
        grid=grid,
        in_specs=[
            pl.BlockSpec((_BLOCK_ROWS, d), lambda i: (i, 0)),
            pl.BlockSpec((_NUM_CLUSTERS, d), lambda i: (0, 0)),
            pl.BlockSpec((1, _NUM_CLUSTERS), lambda i: (0, 0)),
        ],
        out_specs=pl.BlockSpec((_BLOCK_ROWS, d), lambda i: (i, 0)),
        out_shape=jax.ShapeDtypeStruct((n, d), x.dtype),
    )(x, centroids, c2)


# speedup vs baseline: 1.0376x; 1.0376x over previous
import jax
import jax.numpy as jnp
from jax.experimental import pallas as pl

_NUM_CLUSTERS = 8
_EPS = 1e-05
_BLOCK_ROWS = 1024

_SQRT_2_OVER_PI = 0.7978845608028654
_LOG2E = 1.4426950408889634
_BIG = 3.0e38


def _body(x_ref, c_ref, c2_ref, o_ref):
    xb = x_ref[...]
    caug = c_ref[...]
    d = xb.shape[1]

    cdims = (((1,), (1,)), ((), ()))
    dots = jax.lax.dot_general(
        xb, caug, cdims, preferred_element_type=jnp.float32
    )
    s1 = jnp.sum(xb, axis=1, keepdims=True)
    s2 = jnp.sum(xb * xb, axis=1, keepdims=True)

    dist = s2 - 2.0 * dots + c2_ref[...]

    mind = jnp.min(dist, axis=1, keepdims=True)
    lane = jax.lax.broadcasted_iota(jnp.int32, dist.shape, 1)
    lab = jnp.min(
        jnp.where(dist == mind, lane, _NUM_CLUSTERS), axis=1, keepdims=True
    )

    mean = s1 * (1.0 / d)
    var = (s2 - s1 * mean) * (1.0 / (d - 1))
    rstd = jax.lax.rsqrt(var + _EPS)
    xn = (xb - mean) * rstd

    ha = jnp.where(
        lab == 1, _SQRT_2_OVER_PI,
        jnp.where(lab == 2, 1.0, jnp.where((lab == 3) | (lab == 4), 0.5, 0.0)),
    )
    hb = jnp.where(lab == 1, _SQRT_2_OVER_PI * 0.044715, 0.0)
    fp = jnp.where((lab == 1) | (lab == 3), 0.5, 0.0)
    fq = jnp.where(lab == 2, 1.0, jnp.where(lab == 4, 0.5, 0.0))
    ft = jnp.where(lab == 2, -1.0, 0.0)
    is_sp = lab == 7
    is_sig = (lab >= 1) & (lab <= 4)
    lo = jnp.where(is_sp, -_BIG, 0.0)
    up = jnp.where(lab == 5, 6.0, _BIG)
    ek = jnp.where(lab == 6, _LOG2E, jnp.where(is_sp, -_LOG2E, 0.0))

    xnsq = xn * xn
    th = jnp.tanh(xn * (ha + hb * xnsq))
    sig_out = (fp * xn + fq) * (th + 1.0) + ft

    base = jnp.minimum(jnp.maximum(xn, lo), up)
    em = jnp.exp2(ek * xn)
    extra = jnp.where(is_sp, jnp.log(1.0 + em), jnp.minimum(em - 1.0, 0.0))

    out = jnp.where(is_sig, sig_out, base + extra)
    o_ref[...] = out


@jax.jit
def kernel(x, centroids):
    n, d = x.shape
    c2 = jnp.sum(centroids * centroids, axis=-1)[None, :]
    grid = (n // _BLOCK_ROWS,)
    return pl.pallas_call(
        _body,
        grid=grid,
        in_specs=[
            pl.BlockSpec((_BLOCK_ROWS, d), lambda i: (i, 0)),
            pl.BlockSpec((_NUM_CLUSTERS, d), lambda i: (0, 0)),
            pl.BlockSpec((1, _NUM_CLUSTERS), lambda i: (0, 0)),
        ],
        out_specs=pl.BlockSpec((_BLOCK_ROWS, d), lambda i: (i, 0)),
        out_shape=jax.ShapeDtypeStruct((n, d), x.dtype),
    )(x, centroids, c2)


# speedup vs baseline: 1.0377x; 1.0001x over previous
import jax
import jax.numpy as jnp
from jax.experimental import pallas as pl
from jax.experimental.pallas import tpu as pltpu

_NUM_CLUSTERS = 8
_EPS = 1e-05
_BLOCK_ROWS = 1024

_SQRT_2_OVER_PI = 0.7978845608028654
_LOG2E = 1.4426950408889634
_BIG = 3.0e38


def _body(x_ref, c_ref, c2_ref, o_ref):
    xb = x_ref[...]
    caug = c_ref[...]
    d = xb.shape[1]

    cdims = (((1,), (1,)), ((), ()))
    dots = jax.lax.dot_general(
        xb, caug, cdims, preferred_element_type=jnp.float32
    )
    s1 = jnp.sum(xb, axis=1, keepdims=True)
    s2 = jnp.sum(xb * xb, axis=1, keepdims=True)

    dist = s2 - 2.0 * dots + c2_ref[...]

    mind = jnp.min(dist, axis=1, keepdims=True)
    lane = jax.lax.broadcasted_iota(jnp.int32, dist.shape, 1)
    lab = jnp.min(
        jnp.where(dist == mind, lane, _NUM_CLUSTERS), axis=1, keepdims=True
    )

    mean = s1 * (1.0 / d)
    var = (s2 - s1 * mean) * (1.0 / (d - 1))
    rstd = jax.lax.rsqrt(var + _EPS)
    xn = (xb - mean) * rstd

    ha = jnp.where(
        lab == 1, _SQRT_2_OVER_PI,
        jnp.where(lab == 2, 1.0, jnp.where((lab == 3) | (lab == 4), 0.5, 0.0)),
    )
    hb = jnp.where(lab == 1, _SQRT_2_OVER_PI * 0.044715, 0.0)
    fp = jnp.where((lab == 1) | (lab == 3), 0.5, 0.0)
    fq = jnp.where(lab == 2, 1.0, jnp.where(lab == 4, 0.5, 0.0))
    ft = jnp.where(lab == 2, -1.0, 0.0)
    is_sp = lab == 7
    is_sig = (lab >= 1) & (lab <= 4)
    lo = jnp.where(is_sp, -_BIG, 0.0)
    up = jnp.where(lab == 5, 6.0, _BIG)
    ek = jnp.where(lab == 6, _LOG2E, jnp.where(is_sp, -_LOG2E, 0.0))

    xnsq = xn * xn
    th = jnp.tanh(xn * (ha + hb * xnsq))
    sig_out = (fp * xn + fq) * (th + 1.0) + ft

    base = jnp.minimum(jnp.maximum(xn, lo), up)
    em = jnp.exp2(ek * xn)
    extra = jnp.where(is_sp, jnp.log(1.0 + em), jnp.minimum(em - 1.0, 0.0))

    out = jnp.where(is_sig, sig_out, base + extra)
    o_ref[...] = out


@jax.jit
def kernel(x, centroids):
    n, d = x.shape
    c2 = jnp.sum(centroids * centroids, axis=-1)[None, :]
    grid = (n // _BLOCK_ROWS,)
    return pl.pallas_call(
        _body,
        grid=grid,
        in_specs=[
            pl.BlockSpec((_BLOCK_ROWS, d), lambda i: (i, 0)),
            pl.BlockSpec((_NUM_CLUSTERS, d), lambda i: (0, 0)),
            pl.BlockSpec((1, _NUM_CLUSTERS), lambda i: (0, 0)),
        ],
        out_specs=pl.BlockSpec((_BLOCK_ROWS, d), lambda i: (i, 0)),
        out_shape=jax.ShapeDtypeStruct((n, d), x.dtype),
        compiler_params=pltpu.CompilerParams(
            dimension_semantics=("parallel",),
        ),
    )(x, centroids, c2)
